# baseline (speedup 1.0000x reference)
import functools

import jax
import jax.numpy as jnp
from jax import lax
from jax.experimental import pallas as pl
from jax.experimental.pallas import tpu as pltpu
from jax.experimental.pallas import tpu_sc as plsc

VOCAB = 1_000_000
DIM = 64
BATCH = 16384

NUM_CORES = 2
NUM_SUBCORES = 16
NUM_WORKERS = NUM_CORES * NUM_SUBCORES
BPW = BATCH // NUM_WORKERS
CHUNK = 128
NCHUNK = BPW // CHUNK
LANES = 16
GROUPS = BPW // LANES

_mesh = plsc.VectorSubcoreMesh(core_axis_name="c", subcore_axis_name="s")


@functools.partial(
    pl.kernel,
    out_type=jax.ShapeDtypeStruct((BATCH,), jnp.float32),
    mesh=_mesh,
    compiler_params=pltpu.CompilerParams(needs_layout_passes=False,
                                         use_tc_tiling_on_sc=False),
    scratch_types=[
        pltpu.VMEM((BPW,), jnp.int32),
        pltpu.VMEM((BPW,), jnp.int32),
        pltpu.VMEM((BPW, DIM), jnp.float32),
        pltpu.VMEM((BPW, DIM), jnp.float32),
        pltpu.VMEM((BPW,), jnp.float32),
        pltpu.VMEM((BPW,), jnp.float32),
        pltpu.VMEM((BPW,), jnp.float32),
        pltpu.SemaphoreType.DMA,
    ],
)
def _glove_sc(i_hbm, j_hbm, wi_hbm, wj_hbm, bi_hbm, bj_hbm, out_hbm,
              idx_i, idx_j, wi_rows, wj_rows, bi_rows, bj_rows, out_v, sem):
    wid = lax.axis_index("s") * NUM_CORES + lax.axis_index("c")
    base = wid * BPW

    pltpu.sync_copy(i_hbm.at[pl.ds(base, BPW)], idx_i)
    pltpu.sync_copy(j_hbm.at[pl.ds(base, BPW)], idx_j)

    copies = []
    for k in range(NCHUNK):
        s = pl.ds(k * CHUNK, CHUNK)
        copies.append(pltpu.async_copy(wi_hbm.at[idx_i.at[s]], wi_rows.at[s], sem))
        copies.append(pltpu.async_copy(wj_hbm.at[idx_j.at[s]], wj_rows.at[s], sem))
        copies.append(pltpu.async_copy(bi_hbm.at[idx_i.at[s]], bi_rows.at[s], sem))
        copies.append(pltpu.async_copy(bj_hbm.at[idx_j.at[s]], bj_rows.at[s], sem))
    for c in copies:
        c.wait()

    lane = lax.iota(jnp.int32, LANES)

    def group(g, carry):
        rows = g * LANES + lane
        acc = plsc.load_gather(bi_rows, [rows])
        acc = acc + plsc.load_gather(bj_rows, [rows])
        for d in range(DIM):
            dcol = jnp.full((LANES,), d, jnp.int32)
            acc = acc + (plsc.load_gather(wi_rows, [rows, dcol])
                         * plsc.load_gather(wj_rows, [rows, dcol]))
        out_v[pl.ds(g * LANES, LANES)] = acc
        return carry

    lax.fori_loop(0, GROUPS, group, 0)

    pltpu.sync_copy(out_v, out_hbm.at[pl.ds(base, BPW)])


def kernel(i_indices, j_indices, wi, wj, bi, bj):
    return _glove_sc(i_indices.astype(jnp.int32), j_indices.astype(jnp.int32),
                     wi, wj, bi.reshape(VOCAB), bj.reshape(VOCAB))
